# baseline (device time: 82969 ns/iter reference)
import jax
import jax.numpy as jnp
from jax import lax
from jax.experimental import pallas as pl
from jax.experimental.pallas import tpu as pltpu

N_DEV = 8
B_PER = 2
SQ = 256
SKV = 256
HQ = 32
HQ_PER = 4
DH = 64
D_MODEL = 512
HD_PER = HQ_PER * DH
WINDOW = 128
SCALE = 0.125


def kernel(x, Wq, K_ext, V_ext, Wo):
    my = lax.axis_index("i")
    xb = x.astype(jnp.bfloat16)
    wq = Wq.astype(jnp.bfloat16)
    wo = Wo.astype(jnp.bfloat16)
    k_loc = lax.dynamic_slice_in_dim(K_ext, my * B_PER, B_PER, axis=0)
    v_loc = lax.dynamic_slice_in_dim(V_ext, my * B_PER, B_PER, axis=0)
    k_loc = jnp.transpose(k_loc, (0, 2, 1, 3)).astype(jnp.bfloat16)
    v_loc = jnp.transpose(v_loc, (0, 2, 1, 3)).astype(jnp.bfloat16)

    def body(x_ref, wq_ref, k_ref, v_ref, wo_ref, out_ref,
             wq_comm, wo_comm, sq_send, sq_recv, so_send, so_recv):
        my_pos = lax.axis_index("i")
        left = lax.rem(my_pos + N_DEV - 1, N_DEV)
        right = lax.rem(my_pos + 1, N_DEV)

        barrier_sem = pltpu.get_barrier_semaphore()
        for nbr in (left, right):
            pl.semaphore_signal(barrier_sem, inc=1, device_id=(nbr,),
                                device_id_type=pl.DeviceIdType.MESH)
        pl.semaphore_wait(barrier_sem, 2)

        wq_comm[pl.ds(my_pos, 1)] = wq_ref[...].reshape(1, D_MODEL, HD_PER)
        wo_comm[pl.ds(my_pos, 1)] = wo_ref[...].reshape(1, HD_PER, D_MODEL)

        ii = lax.broadcasted_iota(jnp.int32, (SQ, SKV), 0)
        jj = lax.broadcasted_iota(jnp.int32, (SQ, SKV), 1)
        mask = jnp.abs(ii - jj) <= WINDOW
        neg = jnp.float32(-1e9)

        def compute_group(g, wq_g, wo_g, first):
            for b in range(B_PER):
                xrow = x_ref[b]
                q = lax.dot_general(xrow, wq_g, (((1,), (0,)), ((), ())),
                                    preferred_element_type=jnp.float32)
                q = (q * SCALE).astype(jnp.bfloat16)
                ctx_parts = []
                for hh in range(HQ_PER):
                    hidx = g * HQ_PER + hh
                    qh = q[:, hh * DH:(hh + 1) * DH]
                    kh = k_ref[b, pl.ds(hidx, 1), :, :].reshape(SKV, DH)
                    vh = v_ref[b, pl.ds(hidx, 1), :, :].reshape(SKV, DH)
                    s = lax.dot_general(qh, kh, (((1,), (1,)), ((), ())),
                                        preferred_element_type=jnp.float32)
                    s = jnp.where(mask, s, neg)
                    m = jnp.max(s, axis=1, keepdims=True)
                    w = jnp.exp(s - m)
                    w = (w / jnp.sum(w, axis=1, keepdims=True)).astype(jnp.bfloat16)
                    ctx_parts.append(
                        lax.dot_general(w, vh, (((1,), (0,)), ((), ())),
                                        preferred_element_type=jnp.float32
                                        ).astype(jnp.bfloat16))
                ctx = jnp.concatenate(ctx_parts, axis=1)
                part = lax.dot_general(ctx, wo_g, (((1,), (0,)), ((), ())),
                                       preferred_element_type=jnp.float32)
                if first:
                    out_ref[b] = part
                else:
                    out_ref[b] = out_ref[b] + part

        compute_group(my_pos, wq_ref[...], wo_ref[...], first=True)

        for h in range(N_DEV - 1):
            g_fwd = lax.rem(my_pos - h + N_DEV, N_DEV)
            g_in = lax.rem(my_pos - h - 1 + N_DEV, N_DEV)
            rq = pltpu.make_async_remote_copy(
                src_ref=wq_comm.at[g_fwd], dst_ref=wq_comm.at[g_fwd],
                send_sem=sq_send.at[h], recv_sem=sq_recv.at[h],
                device_id=(right,), device_id_type=pl.DeviceIdType.MESH)
            ro = pltpu.make_async_remote_copy(
                src_ref=wo_comm.at[g_fwd], dst_ref=wo_comm.at[g_fwd],
                send_sem=so_send.at[h], recv_sem=so_recv.at[h],
                device_id=(right,), device_id_type=pl.DeviceIdType.MESH)
            rq.start()
            ro.start()
            rq.wait()
            ro.wait()
            wq_g = wq_comm[pl.ds(g_in, 1)].reshape(D_MODEL, HD_PER)
            wo_g = wo_comm[pl.ds(g_in, 1)].reshape(HD_PER, D_MODEL)
            compute_group(g_in, wq_g, wo_g, first=False)

    return pl.pallas_call(
        body,
        out_shape=jax.ShapeDtypeStruct((B_PER, SQ, D_MODEL), jnp.float32),
        in_specs=[pl.BlockSpec(memory_space=pltpu.VMEM)] * 5,
        out_specs=pl.BlockSpec(memory_space=pltpu.VMEM),
        scratch_shapes=[
            pltpu.VMEM((N_DEV, D_MODEL, HD_PER), jnp.bfloat16),
            pltpu.VMEM((N_DEV, HD_PER, D_MODEL), jnp.bfloat16),
            pltpu.SemaphoreType.DMA((N_DEV - 1,)),
            pltpu.SemaphoreType.DMA((N_DEV - 1,)),
            pltpu.SemaphoreType.DMA((N_DEV - 1,)),
            pltpu.SemaphoreType.DMA((N_DEV - 1,)),
        ],
        compiler_params=pltpu.CompilerParams(collective_id=0),
    )(xb, wq, k_loc, v_loc, wo)


# device time: 44072 ns/iter; 1.8826x vs baseline; 1.8826x over previous
import jax
import jax.numpy as jnp
from jax import lax
from jax.experimental import pallas as pl
from jax.experimental.pallas import tpu as pltpu

N_DEV = 8
B_PER = 2
SQ = 256
SKV = 256
HQ = 32
HQ_PER = 4
DH = 64
D_MODEL = 512
HD_PER = HQ_PER * DH
WINDOW = 128
SCALE = 0.125


def kernel(x, Wq, K_ext, V_ext, Wo):
    my = lax.axis_index("i")
    xb = x.astype(jnp.bfloat16)
    wq = Wq.astype(jnp.bfloat16)
    wo = Wo.astype(jnp.bfloat16)
    k_loc = lax.dynamic_slice_in_dim(K_ext, my * B_PER, B_PER, axis=0)
    v_loc = lax.dynamic_slice_in_dim(V_ext, my * B_PER, B_PER, axis=0)
    k_loc = jnp.transpose(k_loc, (0, 2, 1, 3)).astype(jnp.bfloat16)
    v_loc = jnp.transpose(v_loc, (0, 2, 1, 3)).astype(jnp.bfloat16)

    def body(x_ref, wq_ref, k_ref, v_ref, wo_ref, out_ref,
             wq_comm, wo_comm,
             qr_s, qr_r, or_s, or_r, ql_s, ql_r, ol_s, ol_r):
        my_pos = lax.axis_index("i")

        HAM = False

        def perm(p):
            return jnp.where(p < 4, p, 11 - p) if HAM else p

        r_idx = perm(my_pos)

        def pos_at(ring_off):
            return perm(lax.rem(r_idx + ring_off + 2 * N_DEV, N_DEV))

        right = pos_at(1)
        left = pos_at(-1)

        barrier_sem = pltpu.get_barrier_semaphore()
        for nbr in (left, right):
            pl.semaphore_signal(barrier_sem, inc=1, device_id=(nbr,),
                                device_id_type=pl.DeviceIdType.MESH)
        pl.semaphore_wait(barrier_sem, 2)

        wq_comm[pl.ds(my_pos, 1)] = wq_ref[...].reshape(1, D_MODEL, HD_PER)
        wo_comm[pl.ds(my_pos, 1)] = wo_ref[...].reshape(1, HD_PER, D_MODEL)

        ii = lax.broadcasted_iota(jnp.int32, (SQ, SKV), 0)
        jj = lax.broadcasted_iota(jnp.int32, (SQ, SKV), 1)
        mask = jnp.abs(ii - jj) <= WINDOW
        neg = jnp.float32(-1e9)

        def compute_group(g, wq_g, wo_g, first):
            for b in range(B_PER):
                xrow = x_ref[b]
                q = lax.dot_general(xrow, wq_g, (((1,), (0,)), ((), ())),
                                    preferred_element_type=jnp.float32)
                q = (q * SCALE).astype(jnp.bfloat16)
                ctx_parts = []
                for hh in range(HQ_PER):
                    hidx = g * HQ_PER + hh
                    qh = q[:, hh * DH:(hh + 1) * DH]
                    kh = k_ref[b, pl.ds(hidx, 1), :, :].reshape(SKV, DH)
                    vh = v_ref[b, pl.ds(hidx, 1), :, :].reshape(SKV, DH)
                    s = lax.dot_general(qh, kh, (((1,), (1,)), ((), ())),
                                        preferred_element_type=jnp.float32)
                    s = jnp.where(mask, s, neg)
                    m = jnp.max(s, axis=1, keepdims=True)
                    w = jnp.exp(s - m)
                    w = (w / jnp.sum(w, axis=1, keepdims=True)).astype(jnp.bfloat16)
                    ctx_parts.append(
                        lax.dot_general(w, vh, (((1,), (0,)), ((), ())),
                                        preferred_element_type=jnp.float32
                                        ).astype(jnp.bfloat16))
                ctx = jnp.concatenate(ctx_parts, axis=1)
                part = lax.dot_general(ctx, wo_g, (((1,), (0,)), ((), ())),
                                       preferred_element_type=jnp.float32)
                if first:
                    out_ref[b] = part
                else:
                    out_ref[b] = out_ref[b] + part

        R_HOPS = 4
        L_HOPS = 3

        def send_pair(g, dev, sq_send, so_send, sq_recv, so_recv):
            rq = pltpu.make_async_remote_copy(
                src_ref=wq_comm.at[g], dst_ref=wq_comm.at[g],
                send_sem=sq_send, recv_sem=sq_recv,
                device_id=(dev,), device_id_type=pl.DeviceIdType.MESH)
            ro = pltpu.make_async_remote_copy(
                src_ref=wo_comm.at[g], dst_ref=wo_comm.at[g],
                send_sem=so_send, recv_sem=so_recv,
                device_id=(dev,), device_id_type=pl.DeviceIdType.MESH)
            rq.start()
            ro.start()
            return rq, ro

        inflight = []

        inflight += send_pair(my_pos, right, qr_s.at[0], or_s.at[0],
                              qr_r.at[0], or_r.at[0])
        inflight += send_pair(my_pos, left, ql_s.at[0], ol_s.at[0],
                              ql_r.at[0], ol_r.at[0])

        compute_group(my_pos, wq_ref[...], wo_ref[...], first=True)

        def recv_pair(sq_recv, so_recv):
            rq = pltpu.make_async_remote_copy(
                src_ref=wq_comm.at[0], dst_ref=wq_comm.at[0],
                send_sem=sq_recv, recv_sem=sq_recv,
                device_id=(my_pos,), device_id_type=pl.DeviceIdType.MESH)
            ro = pltpu.make_async_remote_copy(
                src_ref=wo_comm.at[0], dst_ref=wo_comm.at[0],
                send_sem=so_recv, recv_sem=so_recv,
                device_id=(my_pos,), device_id_type=pl.DeviceIdType.MESH)
            rq.wait_recv()
            ro.wait_recv()

        for h in range(R_HOPS):
            g_r = pos_at(-(h + 1))
            recv_pair(qr_r.at[h], or_r.at[h])
            if h + 1 < R_HOPS:
                inflight += send_pair(g_r, right, qr_s.at[h + 1],
                                      or_s.at[h + 1], qr_r.at[h + 1],
                                      or_r.at[h + 1])
            if h < L_HOPS:
                g_l = pos_at(h + 1)
                recv_pair(ql_r.at[h], ol_r.at[h])
                if h + 1 < L_HOPS:
                    inflight += send_pair(g_l, left, ql_s.at[h + 1],
                                          ol_s.at[h + 1], ql_r.at[h + 1],
                                          ol_r.at[h + 1])
            wq_g = wq_comm[pl.ds(g_r, 1)].reshape(D_MODEL, HD_PER)
            wo_g = wo_comm[pl.ds(g_r, 1)].reshape(HD_PER, D_MODEL)
            compute_group(g_r, wq_g, wo_g, first=False)
            if h < L_HOPS:
                wq_g = wq_comm[pl.ds(g_l, 1)].reshape(D_MODEL, HD_PER)
                wo_g = wo_comm[pl.ds(g_l, 1)].reshape(HD_PER, D_MODEL)
                compute_group(g_l, wq_g, wo_g, first=False)

        for d in inflight:
            d.wait_send()

    return pl.pallas_call(
        body,
        out_shape=jax.ShapeDtypeStruct((B_PER, SQ, D_MODEL), jnp.float32),
        in_specs=[pl.BlockSpec(memory_space=pltpu.VMEM)] * 5,
        out_specs=pl.BlockSpec(memory_space=pltpu.VMEM),
        scratch_shapes=[
            pltpu.VMEM((N_DEV, D_MODEL, HD_PER), jnp.bfloat16),
            pltpu.VMEM((N_DEV, HD_PER, D_MODEL), jnp.bfloat16),
            pltpu.SemaphoreType.DMA((4,)),
            pltpu.SemaphoreType.DMA((4,)),
            pltpu.SemaphoreType.DMA((4,)),
            pltpu.SemaphoreType.DMA((4,)),
            pltpu.SemaphoreType.DMA((3,)),
            pltpu.SemaphoreType.DMA((3,)),
            pltpu.SemaphoreType.DMA((3,)),
            pltpu.SemaphoreType.DMA((3,)),
        ],
        compiler_params=pltpu.CompilerParams(collective_id=0),
    )(xb, wq, k_loc, v_loc, wo)
